# baseline (speedup 1.0000x reference)
import functools

import jax
import jax.numpy as jnp
from jax import lax
from jax.experimental import pallas as pl
from jax.experimental.pallas import tpu as pltpu
from jax.experimental.pallas import tpu_sc as plsc

N = 10000
E = 320000
D = 128
D2 = 2 * D

NC = 2
NS = 16
NW = NC * NS
L = 16

C = 32
NCHUNK = 316
EPW = C * NCHUNK
E_PAD = EPW * NW
E_ALLOC = E_PAD + 2 * C
N_PAD = 10112
RPT = N_PAD // NS
DEN_W = 16
DEN_R = N_PAD // 16

_INV_SQRT_D = 1.0 / (D ** 0.5)


def _edge_body(q_hbm, kv_hbm, sd_hbm,
               num_out, den_out,
               sdv0, hi0, sc0, qr0, kvr0, orow0, exrow0,
               sdv1, hi1, sc1, qr1, kvr1, orow1, exrow1,
               exbuf, accn, accd,
               semi0, semi1, semg0, semg1, sems0, sems1):
    cid = lax.axis_index("c")
    sid = lax.axis_index("s")
    wid = sid * NC + cid
    ebase = wid * EPW

    zero16 = jnp.zeros((L,), jnp.float32)
    iota16 = lax.iota(jnp.int32, L)
    mask4 = jnp.full((L,), 15, jnp.int32)
    four = jnp.full((L,), 4, jnp.int32)

    sdv = (sdv0, sdv1)
    hi = (hi0, hi1)
    sc = (sc0, sc1)
    qr = (qr0, qr1)
    kvr = (kvr0, kvr1)
    orow = (orow0, orow1)
    exrow = (exrow0, exrow1)
    semi = (semi0, semi1)
    semg = (semg0, semg1)
    sems = (sems0, sems1)

    def _zrow(r, carry):
        for j in range(D // L):
            plsc.store_scatter(orow0, [jnp.full((L,), r, jnp.int32),
                                       iota16 + (j * L)], zero16)
        plsc.store_scatter(exrow0, [jnp.full((L,), r, jnp.int32), iota16],
                           zero16)
        return carry
    lax.fori_loop(jnp.int32(0), jnp.int32(C), _zrow, jnp.int32(0),
                  unroll=False)

    for t in range(RPT // C):
        pltpu.sync_copy(orow0, accn.at[pl.ds(sid * RPT + t * C, C)])
    if RPT % C:
        rem = RPT % C
        pltpu.sync_copy(orow0.at[pl.ds(0, rem)],
                        accn.at[pl.ds(sid * RPT + (RPT // C) * C, rem)])

    @pl.when(sid == 0)
    def _zero_den():
        for t in range(DEN_R // C):
            pltpu.sync_copy(exrow0, accd.at[pl.ds(t * C, C)])
        if DEN_R % C:
            pltpu.sync_copy(exrow0.at[pl.ds(0, DEN_R % C)],
                            accd.at[pl.ds((DEN_R // C) * C, DEN_R % C)])
    plsc.subcore_barrier()

    def issue_idx(j, b):
        pltpu.async_copy(sd_hbm.at[wid * NCHUNK + j], sdv[b], semi[b])

    def drain_idx(b):
        pltpu.make_async_copy(sd_hbm.at[cid * 0], sdv[b], semi[b]).wait()

    def issue_gather(b):
        pltpu.async_copy(q_hbm.at[sdv[b].at[jnp.int32(1)]], qr[b], semg[b])
        pltpu.async_copy(kv_hbm.at[sdv[b].at[jnp.int32(0)]], kvr[b],
                         semg[b])

    def drain_gather(b):
        pltpu.make_async_copy(q_hbm.at[pl.ds(0, C)], qr[b], semg[b]).wait()
        pltpu.make_async_copy(kv_hbm.at[pl.ds(0, C)], kvr[b],
                              semg[b]).wait()

    def issue_scat(b):
        d1 = pltpu.async_copy(orow[b], accn.at[sc[b]], sems[b], add=True)
        d2 = pltpu.async_copy(exrow[b], accd.at[hi[b]], sems[b], add=True)
        return d1, d2

    def compute(b):
        lanes = []
        for g in range(C // L):
            dst16 = sdv[b][1, pl.ds(g * L, L)]
            sc[b][pl.ds(g * L, L)] = dst16
            hi[b][pl.ds(g * L, L)] = lax.shift_right_logical(dst16, four)
            lanes.append(jnp.bitwise_and(dst16, mask4))
        return lanes

    def compute2(b, lanes):
        for g in range(C // L):
            rowi = iota16 + (g * L)
            for i in range(L):
                exrow[b][g * L + i, :] = zero16
            def _dotk(_, carry):
                totv, e = carry
                acc = None
                for j in range(D // L):
                    qv = qr[b][e, pl.ds(j * L, L)]
                    kv = kvr[b][e, jnp.int32(0), pl.ds(j * L, L)]
                    t = qv * kv
                    acc = t if acc is None else acc + t
                tot = jnp.sum(acc)
                totv = jnp.where(iota16 == (e - g * L),
                                 jnp.full((L,), tot), totv)
                return totv, e + 1
            totv, _ = lax.fori_loop(0, L, _dotk,
                                    (zero16, jnp.int32(g * L)), unroll=2)
            exv = jnp.exp(totv * _INV_SQRT_D)
            plsc.store_scatter(exrow[b], [rowi, lanes[g]], exv)
            exbuf[...] = exv

            def _scalek(_, e):
                exb = plsc.load_gather(
                    exbuf, [jnp.full((L,), e - g * L, jnp.int32)])
                for j in range(D // L):
                    vd = kvr[b][e, jnp.int32(1), pl.ds(j * L, L)]
                    orow[b][e, pl.ds(j * L, L)] = vd * exb
                return e + 1
            lax.fori_loop(0, L, _scalek, jnp.int32(g * L), unroll=2)

    def turn(j, b):
        drain_idx(1 - b)
        issue_gather(1 - b)
        lanes = compute(b)
        drain_gather(b)
        issue_idx(j + 2, b)
        compute2(b, lanes)
        return issue_scat(b)

    issue_idx(jnp.int32(0), 0)
    issue_idx(jnp.int32(1), 1)
    drain_idx(0)
    issue_gather(0)
    for d in turn(jnp.int32(0), 0) + turn(jnp.int32(1), 1):
        d.wait()

    def _pair(it, carry):
        j = 2 + 2 * it
        s0 = turn(j, 0)
        s1 = turn(j + 1, 1)
        for d in s0 + s1:
            d.wait()
        return carry
    lax.fori_loop(jnp.int32(0), jnp.int32((NCHUNK - 2) // 2), _pair,
                  jnp.int32(0), unroll=False)

    drain_gather(0)
    drain_idx(1)

    plsc.subcore_barrier()
    pltpu.sync_copy(accn.at[pl.ds(sid * RPT, RPT)],
                    num_out.at[pl.ds(cid * N_PAD + sid * RPT, RPT)])

    @pl.when(sid == 0)
    def _dump_den():
        pltpu.sync_copy(accd, den_out.at[pl.ds(cid * DEN_R, DEN_R)])


def _buf_types():
    return [
        pltpu.VMEM((2, C), jnp.int32),
        pltpu.VMEM((C,), jnp.int32),
        pltpu.VMEM((C,), jnp.int32),
        pltpu.VMEM((C, D), jnp.float32),
        pltpu.VMEM((C, 2, D), jnp.float32),
        pltpu.VMEM((C, D), jnp.float32),
        pltpu.VMEM((C, DEN_W), jnp.float32),
    ]


_edge_kernel = functools.partial(
    pl.kernel,
    out_type=(
        jax.ShapeDtypeStruct((NC * N_PAD, D), jnp.float32),
        jax.ShapeDtypeStruct((NC * DEN_R, DEN_W), jnp.float32),
    ),
    mesh=plsc.VectorSubcoreMesh(core_axis_name="c", subcore_axis_name="s"),
    compiler_params=pltpu.CompilerParams(needs_layout_passes=False),
    scratch_types=_buf_types() + _buf_types() + [
        pltpu.VMEM((L,), jnp.float32),
        pltpu.VMEM_SHARED((N_PAD, D), jnp.float32),
        pltpu.VMEM_SHARED((DEN_R, DEN_W), jnp.float32),
        pltpu.SemaphoreType.DMA,
        pltpu.SemaphoreType.DMA,
        pltpu.SemaphoreType.DMA,
        pltpu.SemaphoreType.DMA,
        pltpu.SemaphoreType.DMA,
        pltpu.SemaphoreType.DMA,
    ],
)(_edge_body)


_RB = 1264


def _qkv_body(x_ref, wq_ref, wk_ref, wv_ref, bq_ref, bk_ref, bv_ref,
              q_ref, kv_ref):
    xb = x_ref[...]
    q_ref[...] = jnp.dot(xb, wq_ref[...],
                         preferred_element_type=jnp.float32) + bq_ref[...]
    kv_ref[:, 0, :] = jnp.dot(xb, wk_ref[...],
                              preferred_element_type=jnp.float32) \
        + bk_ref[...]
    kv_ref[:, 1, :] = jnp.dot(xb, wv_ref[...],
                              preferred_element_type=jnp.float32) \
        + bv_ref[...]


def _qkv(x, wq, wk, wv, bq, bk, bv):
    grid = (N_PAD // _RB,)
    return pl.pallas_call(
        _qkv_body,
        grid=grid,
        in_specs=[
            pl.BlockSpec((_RB, D), lambda i: (i, i * 0)),
            pl.BlockSpec((D, D), lambda i: (i * 0, i * 0)),
            pl.BlockSpec((D, D), lambda i: (i * 0, i * 0)),
            pl.BlockSpec((D, D), lambda i: (i * 0, i * 0)),
            pl.BlockSpec((1, D), lambda i: (i * 0, i * 0)),
            pl.BlockSpec((1, D), lambda i: (i * 0, i * 0)),
            pl.BlockSpec((1, D), lambda i: (i * 0, i * 0)),
        ],
        out_specs=[
            pl.BlockSpec((_RB, D), lambda i: (i, i * 0)),
            pl.BlockSpec((_RB, 2, D), lambda i: (i, i * 0, i * 0)),
        ],
        out_shape=[jax.ShapeDtypeStruct((N_PAD, D), jnp.float32),
                   jax.ShapeDtypeStruct((N_PAD, 2, D), jnp.float32)],
    )(x, wq, wk, wv, bq, bk, bv)


def _epi_body(mode, num_ref, den_ref, h_ref, ws_ref, bs_ref, *rest):
    if mode == 3:
        noise_ref, out_ref = rest
    else:
        (out_ref,) = rest
    nb = num_ref[0] + num_ref[1]
    db = den_ref[0] + den_ref[1]
    hb = h_ref[...]
    o = nb / (db + 1e-16) + jnp.dot(hb, ws_ref[...],
                                    preferred_element_type=jnp.float32) \
        + bs_ref[...]
    if mode == 1:
        out_ref[...] = o * jax.nn.sigmoid(o)
    elif mode == 2:
        o = o + hb
        out_ref[...] = o * jax.nn.sigmoid(o)
    else:
        out_ref[...] = o + noise_ref[...]


def _epilogue(mode, num, den, h, ws, bs, noise=None):
    grid = (N_PAD // _RB,)
    in_specs = [
        pl.BlockSpec((NC, _RB, D), lambda i: (i * 0, i, i * 0)),
        pl.BlockSpec((NC, _RB, 1), lambda i: (i * 0, i, i * 0)),
        pl.BlockSpec((_RB, D), lambda i: (i, i * 0)),
        pl.BlockSpec((D, D), lambda i: (i * 0, i * 0)),
        pl.BlockSpec((1, D), lambda i: (i * 0, i * 0)),
    ]
    args = [num, den, h, ws, bs]
    if mode == 3:
        in_specs.append(pl.BlockSpec((_RB, D), lambda i: (i, i * 0)))
        args.append(noise)
    return pl.pallas_call(
        functools.partial(_epi_body, mode),
        grid=grid,
        in_specs=in_specs,
        out_specs=pl.BlockSpec((_RB, D), lambda i: (i, i * 0)),
        out_shape=jax.ShapeDtypeStruct((N_PAD, D), jnp.float32),
    )(*args)


def kernel(x, edge_index,
           Wq1, Wk1, Wv1, Ws1, bq1, bk1, bv1, bs1,
           Wq2, Wk2, Wv2, Ws2, bq2, bk2, bv2, bs2,
           Wq3, Wk3, Wv3, Ws3, bq3, bk3, bv3, bs3):
    src = edge_index[0].astype(jnp.int32)
    dst = edge_index[1].astype(jnp.int32)
    pad = jnp.full((E_ALLOC - E,), N, jnp.int32)
    src = jnp.concatenate([src, pad])
    dst = jnp.concatenate([dst, pad])
    sd = jnp.stack([src.reshape(E_ALLOC // C, C),
                    dst.reshape(E_ALLOC // C, C)], axis=1)
    xp = jnp.pad(x, ((0, N_PAD - N), (0, 0)))
    noise = 0.1 * jax.random.normal(jax.random.key(123), (N, D),
                                    dtype=jnp.float32)
    noise = jnp.pad(noise, ((0, N_PAD - N), (0, 0)))

    def layer(mode, h, wq, wk, wv, ws, bq, bk, bv, bs):
        q, kv = _qkv(h, wq, wk, wv, bq.reshape(1, D), bk.reshape(1, D),
                     bv.reshape(1, D))
        num, den = _edge_kernel(q, kv, sd)
        num = num.reshape(NC, N_PAD, D)
        den = den.reshape(NC, N_PAD, 1)
        return _epilogue(mode, num, den, h, ws, bs.reshape(1, D),
                         noise if mode == 3 else None)

    h = layer(1, xp, Wq1, Wk1, Wv1, Ws1, bq1, bk1, bv1, bs1)
    h = layer(2, h, Wq2, Wk2, Wv2, Ws2, bq2, bk2, bv2, bs2)
    out = layer(3, h, Wq3, Wk3, Wv3, Ws3, bq3, bk3, bv3, bs3)
    return out[:N]


# speedup vs baseline: 1.0309x; 1.0309x over previous
import functools

import jax
import jax.numpy as jnp
from jax import lax
from jax.experimental import pallas as pl
from jax.experimental.pallas import tpu as pltpu
from jax.experimental.pallas import tpu_sc as plsc

N = 10000
E = 320000
D = 128
D2 = 2 * D

NC = 2
NS = 16
NW = NC * NS
L = 16

C = 32
NCHUNK = 316
EPW = C * NCHUNK
E_PAD = EPW * NW
E_ALLOC = E_PAD + 2 * C
N_PAD = 10112
RPT = N_PAD // NS
DEN_W = 16
DEN_R = N_PAD // 16

_INV_SQRT_D = 1.0 / (D ** 0.5)


def _edge_body(q_hbm, k_hbm, v_hbm, sd_hbm,
               num_out, den_out,
               sdv0, hi0, sc0, qr0, kr0, vr0, orow0, exrow0,
               sdv1, hi1, sc1, qr1, kr1, vr1, orow1, exrow1,
               exbuf, accn, accd,
               semi0, semi1, semg0, semg1, sems0, sems1):
    cid = lax.axis_index("c")
    sid = lax.axis_index("s")
    wid = sid * NC + cid
    ebase = wid * EPW

    zero16 = jnp.zeros((L,), jnp.float32)
    iota16 = lax.iota(jnp.int32, L)
    mask4 = jnp.full((L,), 15, jnp.int32)
    four = jnp.full((L,), 4, jnp.int32)

    sdv = (sdv0, sdv1)
    hi = (hi0, hi1)
    sc = (sc0, sc1)
    qr = (qr0, qr1)
    kr = (kr0, kr1)
    vr = (vr0, vr1)
    orow = (orow0, orow1)
    exrow = (exrow0, exrow1)
    semi = (semi0, semi1)
    semg = (semg0, semg1)
    sems = (sems0, sems1)

    def _zrow(r, carry):
        for j in range(D // L):
            plsc.store_scatter(orow0, [jnp.full((L,), r, jnp.int32),
                                       iota16 + (j * L)], zero16)
        plsc.store_scatter(exrow0, [jnp.full((L,), r, jnp.int32), iota16],
                           zero16)
        return carry
    lax.fori_loop(jnp.int32(0), jnp.int32(C), _zrow, jnp.int32(0),
                  unroll=False)

    for t in range(RPT // C):
        pltpu.sync_copy(orow0, accn.at[pl.ds(sid * RPT + t * C, C)])
    if RPT % C:
        rem = RPT % C
        pltpu.sync_copy(orow0.at[pl.ds(0, rem)],
                        accn.at[pl.ds(sid * RPT + (RPT // C) * C, rem)])

    @pl.when(sid == 0)
    def _zero_den():
        for t in range(DEN_R // C):
            pltpu.sync_copy(exrow0, accd.at[pl.ds(t * C, C)])
        if DEN_R % C:
            pltpu.sync_copy(exrow0.at[pl.ds(0, DEN_R % C)],
                            accd.at[pl.ds((DEN_R // C) * C, DEN_R % C)])
    plsc.subcore_barrier()

    def issue_idx(j, b):
        pltpu.async_copy(sd_hbm.at[wid * NCHUNK + j], sdv[b], semi[b])

    def drain_idx(b):
        pltpu.make_async_copy(sd_hbm.at[cid * 0], sdv[b], semi[b]).wait()

    def issue_gather(b):
        pltpu.async_copy(q_hbm.at[sdv[b].at[jnp.int32(1)]], qr[b], semg[b])
        pltpu.async_copy(k_hbm.at[sdv[b].at[jnp.int32(0)]], kr[b], semg[b])
        pltpu.async_copy(v_hbm.at[sdv[b].at[jnp.int32(0)]], vr[b], semg[b])

    def drain_gather(b):
        pltpu.make_async_copy(q_hbm.at[pl.ds(0, C)], qr[b], semg[b]).wait()
        pltpu.make_async_copy(k_hbm.at[pl.ds(0, C)], kr[b], semg[b]).wait()
        pltpu.make_async_copy(v_hbm.at[pl.ds(0, C)], vr[b], semg[b]).wait()

    def issue_scat(b):
        d1 = pltpu.async_copy(orow[b], accn.at[sc[b]], sems[b], add=True)
        d2 = pltpu.async_copy(exrow[b], accd.at[hi[b]], sems[b], add=True)
        return d1, d2

    def compute(b):
        lanes = []
        for g in range(C // L):
            dst16 = sdv[b][1, pl.ds(g * L, L)]
            sc[b][pl.ds(g * L, L)] = dst16
            hi[b][pl.ds(g * L, L)] = lax.shift_right_logical(dst16, four)
            lanes.append(jnp.bitwise_and(dst16, mask4))
        return lanes

    def compute2(b, lanes):
        for g in range(C // L):
            rowi = iota16 + (g * L)
            for i in range(L):
                exrow[b][g * L + i, :] = zero16
            def _dotk(_, carry):
                totv, e = carry
                acc = None
                for j in range(D // L):
                    qv = qr[b][e, pl.ds(j * L, L)]
                    kv = kr[b][e, pl.ds(j * L, L)]
                    t = qv * kv
                    acc = t if acc is None else acc + t
                tot = jnp.sum(acc)
                totv = jnp.where(iota16 == (e - g * L),
                                 jnp.full((L,), tot), totv)
                return totv, e + 1
            totv, _ = lax.fori_loop(0, L, _dotk,
                                    (zero16, jnp.int32(g * L)), unroll=2)
            exv = jnp.exp(totv * _INV_SQRT_D)
            plsc.store_scatter(exrow[b], [rowi, lanes[g]], exv)
            exbuf[...] = exv

            def _scalek(_, e):
                exb = plsc.load_gather(
                    exbuf, [jnp.full((L,), e - g * L, jnp.int32)])
                for j in range(D // L):
                    vd = vr[b][e, pl.ds(j * L, L)]
                    orow[b][e, pl.ds(j * L, L)] = vd * exb
                return e + 1
            lax.fori_loop(0, L, _scalek, jnp.int32(g * L), unroll=2)

    def turn(j, b):
        drain_idx(1 - b)
        issue_gather(1 - b)
        lanes = compute(b)
        drain_gather(b)
        issue_idx(j + 2, b)
        compute2(b, lanes)
        return issue_scat(b)

    issue_idx(jnp.int32(0), 0)
    issue_idx(jnp.int32(1), 1)
    drain_idx(0)
    issue_gather(0)
    for d in turn(jnp.int32(0), 0) + turn(jnp.int32(1), 1):
        d.wait()

    def _pair(it, j):
        s0 = turn(j, 0)
        s1 = turn(j + 1, 1)
        for d in s0 + s1:
            d.wait()
        return j + 2
    lax.fori_loop(0, (NCHUNK - 2) // 2, _pair, jnp.int32(2), unroll=2)

    drain_gather(0)
    drain_idx(1)

    plsc.subcore_barrier()
    pltpu.sync_copy(accn.at[pl.ds(sid * RPT, RPT)],
                    num_out.at[pl.ds(cid * N_PAD + sid * RPT, RPT)])

    @pl.when(sid == 0)
    def _dump_den():
        pltpu.sync_copy(accd, den_out.at[pl.ds(cid * DEN_R, DEN_R)])


def _buf_types():
    return [
        pltpu.VMEM((2, C), jnp.int32),
        pltpu.VMEM((C,), jnp.int32),
        pltpu.VMEM((C,), jnp.int32),
        pltpu.VMEM((C, D), jnp.float32),
        pltpu.VMEM((C, D), jnp.float32),
        pltpu.VMEM((C, D), jnp.float32),
        pltpu.VMEM((C, D), jnp.float32),
        pltpu.VMEM((C, DEN_W), jnp.float32),
    ]


_edge_kernel = functools.partial(
    pl.kernel,
    out_type=(
        jax.ShapeDtypeStruct((NC * N_PAD, D), jnp.float32),
        jax.ShapeDtypeStruct((NC * DEN_R, DEN_W), jnp.float32),
    ),
    mesh=plsc.VectorSubcoreMesh(core_axis_name="c", subcore_axis_name="s"),
    compiler_params=pltpu.CompilerParams(needs_layout_passes=False),
    scratch_types=_buf_types() + _buf_types() + [
        pltpu.VMEM((L,), jnp.float32),
        pltpu.VMEM_SHARED((N_PAD, D), jnp.float32),
        pltpu.VMEM_SHARED((DEN_R, DEN_W), jnp.float32),
        pltpu.SemaphoreType.DMA,
        pltpu.SemaphoreType.DMA,
        pltpu.SemaphoreType.DMA,
        pltpu.SemaphoreType.DMA,
        pltpu.SemaphoreType.DMA,
        pltpu.SemaphoreType.DMA,
    ],
)(_edge_body)


_RB = 1264


def _qkv_body(x_ref, wq_ref, wk_ref, wv_ref, bq_ref, bk_ref, bv_ref,
              q_ref, k_ref, v_ref):
    xb = x_ref[...]
    q_ref[...] = jnp.dot(xb, wq_ref[...],
                         preferred_element_type=jnp.float32) + bq_ref[...]
    k_ref[...] = jnp.dot(xb, wk_ref[...],
                         preferred_element_type=jnp.float32) + bk_ref[...]
    v_ref[...] = jnp.dot(xb, wv_ref[...],
                         preferred_element_type=jnp.float32) + bv_ref[...]


def _qkv(x, wq, wk, wv, bq, bk, bv):
    grid = (N_PAD // _RB,)
    return pl.pallas_call(
        _qkv_body,
        grid=grid,
        in_specs=[
            pl.BlockSpec((_RB, D), lambda i: (i, i * 0)),
            pl.BlockSpec((D, D), lambda i: (i * 0, i * 0)),
            pl.BlockSpec((D, D), lambda i: (i * 0, i * 0)),
            pl.BlockSpec((D, D), lambda i: (i * 0, i * 0)),
            pl.BlockSpec((1, D), lambda i: (i * 0, i * 0)),
            pl.BlockSpec((1, D), lambda i: (i * 0, i * 0)),
            pl.BlockSpec((1, D), lambda i: (i * 0, i * 0)),
        ],
        out_specs=[
            pl.BlockSpec((_RB, D), lambda i: (i, i * 0)),
            pl.BlockSpec((_RB, D), lambda i: (i, i * 0)),
            pl.BlockSpec((_RB, D), lambda i: (i, i * 0)),
        ],
        out_shape=[jax.ShapeDtypeStruct((N_PAD, D), jnp.float32)] * 3,
    )(x, wq, wk, wv, bq, bk, bv)


def _epi_body(mode, num_ref, den_ref, h_ref, ws_ref, bs_ref, *rest):
    if mode == 3:
        noise_ref, out_ref = rest
    else:
        (out_ref,) = rest
    nb = num_ref[0] + num_ref[1]
    db = den_ref[0] + den_ref[1]
    hb = h_ref[...]
    o = nb / (db + 1e-16) + jnp.dot(hb, ws_ref[...],
                                    preferred_element_type=jnp.float32) \
        + bs_ref[...]
    if mode == 1:
        out_ref[...] = o * jax.nn.sigmoid(o)
    elif mode == 2:
        o = o + hb
        out_ref[...] = o * jax.nn.sigmoid(o)
    else:
        out_ref[...] = o + noise_ref[...]


def _epilogue(mode, num, den, h, ws, bs, noise=None):
    grid = (N_PAD // _RB,)
    in_specs = [
        pl.BlockSpec((NC, _RB, D), lambda i: (i * 0, i, i * 0)),
        pl.BlockSpec((NC, _RB, 1), lambda i: (i * 0, i, i * 0)),
        pl.BlockSpec((_RB, D), lambda i: (i, i * 0)),
        pl.BlockSpec((D, D), lambda i: (i * 0, i * 0)),
        pl.BlockSpec((1, D), lambda i: (i * 0, i * 0)),
    ]
    args = [num, den, h, ws, bs]
    if mode == 3:
        in_specs.append(pl.BlockSpec((_RB, D), lambda i: (i, i * 0)))
        args.append(noise)
    return pl.pallas_call(
        functools.partial(_epi_body, mode),
        grid=grid,
        in_specs=in_specs,
        out_specs=pl.BlockSpec((_RB, D), lambda i: (i, i * 0)),
        out_shape=jax.ShapeDtypeStruct((N_PAD, D), jnp.float32),
    )(*args)


def kernel(x, edge_index,
           Wq1, Wk1, Wv1, Ws1, bq1, bk1, bv1, bs1,
           Wq2, Wk2, Wv2, Ws2, bq2, bk2, bv2, bs2,
           Wq3, Wk3, Wv3, Ws3, bq3, bk3, bv3, bs3):
    src = edge_index[0].astype(jnp.int32)
    dst = edge_index[1].astype(jnp.int32)
    pad = jnp.full((E_ALLOC - E,), N, jnp.int32)
    src = jnp.concatenate([src, pad])
    dst = jnp.concatenate([dst, pad])
    sd = jnp.stack([src.reshape(E_ALLOC // C, C),
                    dst.reshape(E_ALLOC // C, C)], axis=1)
    xp = jnp.pad(x, ((0, N_PAD - N), (0, 0)))
    noise = 0.1 * jax.random.normal(jax.random.key(123), (N, D),
                                    dtype=jnp.float32)
    noise = jnp.pad(noise, ((0, N_PAD - N), (0, 0)))

    def layer(mode, h, wq, wk, wv, ws, bq, bk, bv, bs):
        q, k, v = _qkv(h, wq, wk, wv, bq.reshape(1, D), bk.reshape(1, D),
                       bv.reshape(1, D))
        num, den = _edge_kernel(q, k, v, sd)
        num = num.reshape(NC, N_PAD, D)
        den = den.reshape(NC, N_PAD, 1)
        return _epilogue(mode, num, den, h, ws, bs.reshape(1, D),
                         noise if mode == 3 else None)

    h = layer(1, xp, Wq1, Wk1, Wv1, Ws1, bq1, bk1, bv1, bs1)
    h = layer(2, h, Wq2, Wk2, Wv2, Ws2, bq2, bk2, bv2, bs2)
    out = layer(3, h, Wq3, Wk3, Wv3, Ws3, bq3, bk3, bv3, bs3)
    return out[:N]


# speedup vs baseline: 1.0735x; 1.0412x over previous
import functools

import jax
import jax.numpy as jnp
from jax import lax
from jax.experimental import pallas as pl
from jax.experimental.pallas import tpu as pltpu
from jax.experimental.pallas import tpu_sc as plsc

N = 10000
E = 320000
D = 128
D2 = 2 * D

NC = 2
NS = 16
NW = NC * NS
L = 16

C = 32
NCHUNK = 316
EPW = C * NCHUNK
E_PAD = EPW * NW
E_ALLOC = E_PAD + 2 * C
N_PAD = 10112
RPT = N_PAD // NS
DEN_W = 16
DEN_R = N_PAD // 16

_INV_SQRT_D = 1.0 / (D ** 0.5)


def _edge_body(q_hbm, k_hbm, v_hbm, sd_hbm,
               num_out, den_out,
               sdv0, hi0, sc0, qr0, kr0, vr0, orow0, exrow0,
               sdv1, hi1, sc1, qr1, kr1, vr1, orow1, exrow1,
               exbuf, accn, accd,
               semi0, semi1, semg0, semg1, sems0, sems1):
    cid = lax.axis_index("c")
    sid = lax.axis_index("s")
    wid = sid * NC + cid
    ebase = wid * EPW

    zero16 = jnp.zeros((L,), jnp.float32)
    iota16 = lax.iota(jnp.int32, L)
    mask4 = jnp.full((L,), 15, jnp.int32)
    four = jnp.full((L,), 4, jnp.int32)

    sdv = (sdv0, sdv1)
    hi = (hi0, hi1)
    sc = (sc0, sc1)
    qr = (qr0, qr1)
    kr = (kr0, kr1)
    vr = (vr0, vr1)
    orow = (orow0, orow1)
    exrow = (exrow0, exrow1)
    semi = (semi0, semi1)
    semg = (semg0, semg1)
    sems = (sems0, sems1)

    def _zrow(r, carry):
        for j in range(D // L):
            plsc.store_scatter(orow0, [jnp.full((L,), r, jnp.int32),
                                       iota16 + (j * L)], zero16)
        plsc.store_scatter(exrow0, [jnp.full((L,), r, jnp.int32), iota16],
                           zero16)
        return carry
    lax.fori_loop(jnp.int32(0), jnp.int32(C), _zrow, jnp.int32(0),
                  unroll=False)

    for t in range(RPT // C):
        pltpu.sync_copy(orow0, accn.at[pl.ds(sid * RPT + t * C, C)])
    if RPT % C:
        rem = RPT % C
        pltpu.sync_copy(orow0.at[pl.ds(0, rem)],
                        accn.at[pl.ds(sid * RPT + (RPT // C) * C, rem)])

    @pl.when(sid == 0)
    def _zero_den():
        for t in range(DEN_R // C):
            pltpu.sync_copy(exrow0, accd.at[pl.ds(t * C, C)])
        if DEN_R % C:
            pltpu.sync_copy(exrow0.at[pl.ds(0, DEN_R % C)],
                            accd.at[pl.ds((DEN_R // C) * C, DEN_R % C)])
    plsc.subcore_barrier()

    def issue_idx(j, b):
        pltpu.async_copy(sd_hbm.at[wid * NCHUNK + j], sdv[b], semi[b])

    def drain_idx(b):
        pltpu.make_async_copy(sd_hbm.at[cid * 0], sdv[b], semi[b]).wait()

    def issue_gather(b):
        pltpu.async_copy(q_hbm.at[sdv[b].at[jnp.int32(1)]], qr[b], semg[b])
        pltpu.async_copy(k_hbm.at[sdv[b].at[jnp.int32(0)]], kr[b], semg[b])
        pltpu.async_copy(v_hbm.at[sdv[b].at[jnp.int32(0)]], vr[b], semg[b])

    def drain_gather(b):
        pltpu.make_async_copy(q_hbm.at[pl.ds(0, C)], qr[b], semg[b]).wait()
        pltpu.make_async_copy(k_hbm.at[pl.ds(0, C)], kr[b], semg[b]).wait()
        pltpu.make_async_copy(v_hbm.at[pl.ds(0, C)], vr[b], semg[b]).wait()

    def issue_scat(b):
        d1 = pltpu.async_copy(orow[b], accn.at[sc[b]], sems[b], add=True)
        d2 = pltpu.async_copy(exrow[b], accd.at[hi[b]], sems[b], add=True)
        return d1, d2

    def compute(b):
        lanes = []
        for g in range(C // L):
            dst16 = sdv[b][1, pl.ds(g * L, L)]
            sc[b][pl.ds(g * L, L)] = dst16
            hi[b][pl.ds(g * L, L)] = lax.shift_right_logical(dst16, four)
            lanes.append(jnp.bitwise_and(dst16, mask4))
        return lanes

    def compute2(b, lanes):
        for g in range(C // L):
            rowi = iota16 + (g * L)
            for i in range(L):
                exrow[b][g * L + i, :] = zero16
            def _dotk(_, carry):
                totv, e = carry
                acc = None
                for j in range(D // L):
                    qv = qr[b][e, pl.ds(j * L, L)]
                    kv = kr[b][e, pl.ds(j * L, L)]
                    t = qv * kv
                    acc = t if acc is None else acc + t
                tot = jnp.sum(acc)
                totv = jnp.where(iota16 == (e - g * L),
                                 jnp.full((L,), tot), totv)
                return totv, e + 1
            totv, _ = lax.fori_loop(0, L, _dotk,
                                    (zero16, jnp.int32(g * L)), unroll=2)
            exv = jnp.exp(totv * _INV_SQRT_D)
            plsc.store_scatter(exrow[b], [rowi, lanes[g]], exv)
            exbuf[...] = exv

            def _scalek(_, e):
                exb = plsc.load_gather(
                    exbuf, [jnp.full((L,), e - g * L, jnp.int32)])
                for j in range(D // L):
                    vd = vr[b][e, pl.ds(j * L, L)]
                    orow[b][e, pl.ds(j * L, L)] = vd * exb
                return e + 1
            lax.fori_loop(0, L, _scalek, jnp.int32(g * L), unroll=2)

    def turn(j, b):
        drain_idx(1 - b)
        issue_gather(1 - b)
        lanes = compute(b)
        drain_gather(b)
        issue_idx(j + 2, b)
        compute2(b, lanes)
        return issue_scat(b)

    issue_idx(jnp.int32(0), 0)
    issue_idx(jnp.int32(1), 1)
    drain_idx(0)
    issue_gather(0)
    for d in turn(jnp.int32(0), 0) + turn(jnp.int32(1), 1):
        d.wait()

    def _pair(it, carry):
        j = 2 + 2 * it
        s0 = turn(j, 0)
        s1 = turn(j + 1, 1)
        for d in s0 + s1:
            d.wait()
        return carry
    lax.fori_loop(jnp.int32(0), jnp.int32((NCHUNK - 2) // 2), _pair,
                  jnp.int32(0), unroll=False)

    drain_gather(0)
    drain_idx(1)

    plsc.subcore_barrier()
    pltpu.sync_copy(accn.at[pl.ds(sid * RPT, RPT)],
                    num_out.at[pl.ds(cid * N_PAD + sid * RPT, RPT)])

    @pl.when(sid == 0)
    def _dump_den():
        pltpu.sync_copy(accd, den_out.at[pl.ds(cid * DEN_R, DEN_R)])


def _buf_types():
    return [
        pltpu.VMEM((2, C), jnp.int32),
        pltpu.VMEM((C,), jnp.int32),
        pltpu.VMEM((C,), jnp.int32),
        pltpu.VMEM((C, D), jnp.float32),
        pltpu.VMEM((C, D), jnp.float32),
        pltpu.VMEM((C, D), jnp.float32),
        pltpu.VMEM((C, D), jnp.float32),
        pltpu.VMEM((C, DEN_W), jnp.float32),
    ]


_edge_kernel = functools.partial(
    pl.kernel,
    out_type=(
        jax.ShapeDtypeStruct((NC * N_PAD, D), jnp.float32),
        jax.ShapeDtypeStruct((NC * DEN_R, DEN_W), jnp.float32),
    ),
    mesh=plsc.VectorSubcoreMesh(core_axis_name="c", subcore_axis_name="s"),
    compiler_params=pltpu.CompilerParams(needs_layout_passes=False),
    scratch_types=_buf_types() + _buf_types() + [
        pltpu.VMEM((L,), jnp.float32),
        pltpu.VMEM_SHARED((N_PAD, D), jnp.float32),
        pltpu.VMEM_SHARED((DEN_R, DEN_W), jnp.float32),
        pltpu.SemaphoreType.DMA,
        pltpu.SemaphoreType.DMA,
        pltpu.SemaphoreType.DMA,
        pltpu.SemaphoreType.DMA,
        pltpu.SemaphoreType.DMA,
        pltpu.SemaphoreType.DMA,
    ],
)(_edge_body)


_RB = 1264


def _qkv_body(x_ref, wq_ref, wk_ref, wv_ref, bq_ref, bk_ref, bv_ref,
              q_ref, k_ref, v_ref):
    xb = x_ref[...]
    q_ref[...] = jnp.dot(xb, wq_ref[...],
                         preferred_element_type=jnp.float32) + bq_ref[...]
    k_ref[...] = jnp.dot(xb, wk_ref[...],
                         preferred_element_type=jnp.float32) + bk_ref[...]
    v_ref[...] = jnp.dot(xb, wv_ref[...],
                         preferred_element_type=jnp.float32) + bv_ref[...]


def _qkv(x, wq, wk, wv, bq, bk, bv):
    grid = (N_PAD // _RB,)
    return pl.pallas_call(
        _qkv_body,
        grid=grid,
        in_specs=[
            pl.BlockSpec((_RB, D), lambda i: (i, i * 0)),
            pl.BlockSpec((D, D), lambda i: (i * 0, i * 0)),
            pl.BlockSpec((D, D), lambda i: (i * 0, i * 0)),
            pl.BlockSpec((D, D), lambda i: (i * 0, i * 0)),
            pl.BlockSpec((1, D), lambda i: (i * 0, i * 0)),
            pl.BlockSpec((1, D), lambda i: (i * 0, i * 0)),
            pl.BlockSpec((1, D), lambda i: (i * 0, i * 0)),
        ],
        out_specs=[
            pl.BlockSpec((_RB, D), lambda i: (i, i * 0)),
            pl.BlockSpec((_RB, D), lambda i: (i, i * 0)),
            pl.BlockSpec((_RB, D), lambda i: (i, i * 0)),
        ],
        out_shape=[jax.ShapeDtypeStruct((N_PAD, D), jnp.float32)] * 3,
    )(x, wq, wk, wv, bq, bk, bv)


def _epi_body(mode, num_ref, den_ref, h_ref, ws_ref, bs_ref, *rest):
    if mode == 3:
        noise_ref, out_ref = rest
    else:
        (out_ref,) = rest
    nb = num_ref[0] + num_ref[1]
    db = den_ref[0] + den_ref[1]
    hb = h_ref[...]
    o = nb / (db + 1e-16) + jnp.dot(hb, ws_ref[...],
                                    preferred_element_type=jnp.float32) \
        + bs_ref[...]
    if mode == 1:
        out_ref[...] = o * jax.nn.sigmoid(o)
    elif mode == 2:
        o = o + hb
        out_ref[...] = o * jax.nn.sigmoid(o)
    else:
        out_ref[...] = o + noise_ref[...]


def _epilogue(mode, num, den, h, ws, bs, noise=None):
    grid = (N_PAD // _RB,)
    in_specs = [
        pl.BlockSpec((NC, _RB, D), lambda i: (i * 0, i, i * 0)),
        pl.BlockSpec((NC, _RB, 1), lambda i: (i * 0, i, i * 0)),
        pl.BlockSpec((_RB, D), lambda i: (i, i * 0)),
        pl.BlockSpec((D, D), lambda i: (i * 0, i * 0)),
        pl.BlockSpec((1, D), lambda i: (i * 0, i * 0)),
    ]
    args = [num, den, h, ws, bs]
    if mode == 3:
        in_specs.append(pl.BlockSpec((_RB, D), lambda i: (i, i * 0)))
        args.append(noise)
    return pl.pallas_call(
        functools.partial(_epi_body, mode),
        grid=grid,
        in_specs=in_specs,
        out_specs=pl.BlockSpec((_RB, D), lambda i: (i, i * 0)),
        out_shape=jax.ShapeDtypeStruct((N_PAD, D), jnp.float32),
    )(*args)


def kernel(x, edge_index,
           Wq1, Wk1, Wv1, Ws1, bq1, bk1, bv1, bs1,
           Wq2, Wk2, Wv2, Ws2, bq2, bk2, bv2, bs2,
           Wq3, Wk3, Wv3, Ws3, bq3, bk3, bv3, bs3):
    src = edge_index[0].astype(jnp.int32)
    dst = edge_index[1].astype(jnp.int32)
    pad = jnp.full((E_ALLOC - E,), N, jnp.int32)
    src = jnp.concatenate([src, pad])
    dst = jnp.concatenate([dst, pad])
    sd = jnp.stack([src.reshape(E_ALLOC // C, C),
                    dst.reshape(E_ALLOC // C, C)], axis=1)
    xp = jnp.pad(x, ((0, N_PAD - N), (0, 0)))
    noise = 0.1 * jax.random.normal(jax.random.key(123), (N, D),
                                    dtype=jnp.float32)
    noise = jnp.pad(noise, ((0, N_PAD - N), (0, 0)))

    def layer(mode, h, wq, wk, wv, ws, bq, bk, bv, bs):
        q, k, v = _qkv(h, wq, wk, wv, bq.reshape(1, D), bk.reshape(1, D),
                       bv.reshape(1, D))
        num, den = _edge_kernel(q, k, v, sd)
        num = num.reshape(NC, N_PAD, D)
        den = den.reshape(NC, N_PAD, 1)
        return _epilogue(mode, num, den, h, ws, bs.reshape(1, D),
                         noise if mode == 3 else None)

    h = layer(1, xp, Wq1, Wk1, Wv1, Ws1, bq1, bk1, bv1, bs1)
    h = layer(2, h, Wq2, Wk2, Wv2, Ws2, bq2, bk2, bv2, bs2)
    out = layer(3, h, Wq3, Wk3, Wv3, Ws3, bq3, bk3, bv3, bs3)
    return out[:N]
